# baseline (speedup 1.0000x reference)
import functools

import numpy as np
import jax
import jax.numpy as jnp
from jax.experimental import pallas as pl
from jax.experimental.pallas import tpu as pltpu

N, D, HD, H = 50000, 256, 1024, 4
RATIO = 0.5
TILE = 1000

R, C = 512, 128
M = R * C
_LANE_JS = [1 << t for t in range(7)]
_ROW_JS = [1 << t for t in range(9)]

_ar_c = np.arange(C)
_P_LANE = np.stack([(_ar_c[:, None] ^ j) == _ar_c[None, :] for j in _LANE_JS]).astype(np.float32)
_ar_r = np.arange(R)
_P_ROW = np.stack([(_ar_r[:, None] ^ j) == _ar_r[None, :] for j in _ROW_JS]).astype(np.float32)


def _score_body(x_ref, w1_ref, b1_ref, w2_ref, b2_ref, out_ref):
    h = jnp.dot(x_ref[...], w1_ref[...]) + b1_ref[...]
    h = jnp.where(h >= 0, h, 0.2 * h)
    out_ref[...] = jnp.dot(h, w2_ref[...]) + b2_ref[...]


def _attn4(x, W1, b1, W2, b2):
    grid = (N // TILE,)
    return pl.pallas_call(
        _score_body,
        grid=grid,
        in_specs=[
            pl.BlockSpec((TILE, D), lambda i: (i, 0)),
            pl.BlockSpec((D, HD), lambda i: (0, 0)),
            pl.BlockSpec((1, HD), lambda i: (0, 0)),
            pl.BlockSpec((HD, H), lambda i: (0, 0)),
            pl.BlockSpec((1, H), lambda i: (0, 0)),
        ],
        out_specs=pl.BlockSpec((TILE, H), lambda i: (i, 0)),
        out_shape=jax.ShapeDtypeStruct((N, H), jnp.float32),
    )(x, W1, b1.reshape(1, HD), W2, b2.reshape(1, H))


def _sort_body(s_ref, i_ref, p_lane_ref, p_row_ref, ss_ref, si_ref):
    s = s_ref[...]
    i = i_ref[...]
    r_iota = jax.lax.broadcasted_iota(jnp.int32, (R, C), 0)
    c_iota = jax.lax.broadcasted_iota(jnp.int32, (R, C), 1)
    k = 2
    while k <= M:
        j = k // 2
        while j >= 1:
            if j < C:
                pi = _LANE_JS.index(j)
                sp = jnp.dot(s, p_lane_ref[pi], precision=jax.lax.Precision.HIGHEST)
                ip = jnp.dot(i, p_lane_ref[pi], precision=jax.lax.Precision.HIGHEST)
                ej = (c_iota & j) == 0
            else:
                jr = j // C
                pi = _ROW_JS.index(jr)
                sp = jnp.dot(p_row_ref[pi], s, precision=jax.lax.Precision.HIGHEST)
                ip = jnp.dot(p_row_ref[pi], i, precision=jax.lax.Precision.HIGHEST)
                ej = (r_iota & jr) == 0
            if k < C:
                up = (c_iota & k) == 0
            elif k < M:
                up = (r_iota & (k // C)) == 0
            else:
                up = jnp.full((R, C), True)
            lt = (s > sp) | ((s == sp) & (i < ip))
            keep_e = lt == (up == ej)
            s = jnp.where(keep_e, s, sp)
            i = jnp.where(keep_e, i, ip)
            j //= 2
        k *= 2
    ss_ref[...] = s
    si_ref[...] = i.astype(jnp.int32)


def _topk_sort(scores):
    s2 = jnp.concatenate([scores, jnp.full((M - N,), -1.0, jnp.float32)]).reshape(R, C)
    i2 = jnp.arange(M, dtype=jnp.float32).reshape(R, C)
    ss, si = pl.pallas_call(
        _sort_body,
        out_shape=(
            jax.ShapeDtypeStruct((R, C), jnp.float32),
            jax.ShapeDtypeStruct((R, C), jnp.int32),
        ),
    )(s2, i2, jnp.asarray(_P_LANE), jnp.asarray(_P_ROW))
    return ss.reshape(M), si.reshape(M)


def kernel(x, W1, b1, W2, b2):
    attn4 = _attn4(x, W1, b1, W2, b2)
    attn = attn4.mean(axis=1)
    scores = jax.nn.sigmoid(attn)
    k = max(1, int(RATIO * N))
    sorted_scores, sorted_idx = _topk_sort(scores)
    idx = sorted_idx[:k]
    node_feat = jnp.take(x, idx, axis=0)
    scaled_feat = node_feat * (1.0 + sorted_scores[:k][:, None])
    return (scaled_feat, idx, scores)


# speedup vs baseline: 1.1676x; 1.1676x over previous
import functools

import numpy as np
import jax
import jax.numpy as jnp
from jax.experimental import pallas as pl
from jax.experimental.pallas import tpu as pltpu

N, D, HD, H = 50000, 256, 1024, 4
RATIO = 0.5
TILE = 1000

R, C = 512, 128
M = R * C
_LANE_JS = [1 << t for t in range(7)]
_ROW_JS = [1 << t for t in range(9)]

_ar_c = np.arange(C)
_P_LANE = np.stack([(_ar_c[:, None] ^ j) == _ar_c[None, :] for j in _LANE_JS]).astype(np.float32)
_ar_r = np.arange(R)
_P_ROW = np.stack([(_ar_r[:, None] ^ j) == _ar_r[None, :] for j in _ROW_JS]).astype(np.float32)


def _score_body(x_ref, w1_ref, b1_ref, w2_ref, b2_ref, out_ref):
    h = jnp.dot(x_ref[...], w1_ref[...]) + b1_ref[...]
    h = jnp.where(h >= 0, h, 0.2 * h)
    out_ref[...] = jnp.dot(h, w2_ref[...]) + b2_ref[...]


def _attn4(x, W1, b1, W2, b2):
    grid = (N // TILE,)
    return pl.pallas_call(
        _score_body,
        grid=grid,
        in_specs=[
            pl.BlockSpec((TILE, D), lambda i: (i, 0)),
            pl.BlockSpec((D, HD), lambda i: (0, 0)),
            pl.BlockSpec((1, HD), lambda i: (0, 0)),
            pl.BlockSpec((HD, H), lambda i: (0, 0)),
            pl.BlockSpec((1, H), lambda i: (0, 0)),
        ],
        out_specs=pl.BlockSpec((TILE, H), lambda i: (i, 0)),
        out_shape=jax.ShapeDtypeStruct((N, H), jnp.float32),
    )(x, W1, b1.reshape(1, HD), W2, b2.reshape(1, H))


def _sort_body(s_ref, i_ref, p_lane_ref, ss_ref, si_ref):
    s = s_ref[...]
    i = i_ref[...]
    r_iota = jax.lax.broadcasted_iota(jnp.int32, (R, C), 0)
    c_iota = jax.lax.broadcasted_iota(jnp.int32, (R, C), 1)
    k = 2
    while k <= M:
        j = k // 2
        while j >= 1:
            if j < C:
                pi = _LANE_JS.index(j)
                both = jnp.concatenate([s, i], axis=0)
                bp = jnp.dot(both, p_lane_ref[pi], precision=jax.lax.Precision.HIGHEST)
                sp, ip = bp[:R], bp[R:]
                ej = (c_iota & j) == 0
            else:
                jr = j // C
                ej = (r_iota & jr) == 0
                s_up = jnp.concatenate([s[jr:], s[:jr]], axis=0)
                s_dn = jnp.concatenate([s[-jr:], s[:-jr]], axis=0)
                sp = jnp.where(ej, s_up, s_dn)
                i_up = jnp.concatenate([i[jr:], i[:jr]], axis=0)
                i_dn = jnp.concatenate([i[-jr:], i[:-jr]], axis=0)
                ip = jnp.where(ej, i_up, i_dn)
            if k < C:
                up = (c_iota & k) == 0
            elif k < M:
                up = (r_iota & (k // C)) == 0
            else:
                up = jnp.full((R, C), True)
            lt = (s > sp) | ((s == sp) & (i < ip))
            keep_e = lt == (up == ej)
            s = jnp.where(keep_e, s, sp)
            i = jnp.where(keep_e, i, ip)
            j //= 2
        k *= 2
    ss_ref[...] = s
    si_ref[...] = i.astype(jnp.int32)


def _topk_sort(scores):
    s2 = jnp.concatenate([scores, jnp.full((M - N,), -1.0, jnp.float32)]).reshape(R, C)
    i2 = jnp.arange(M, dtype=jnp.float32).reshape(R, C)
    ss, si = pl.pallas_call(
        _sort_body,
        out_shape=(
            jax.ShapeDtypeStruct((R, C), jnp.float32),
            jax.ShapeDtypeStruct((R, C), jnp.int32),
        ),
    )(s2, i2, jnp.asarray(_P_LANE))
    return ss.reshape(M), si.reshape(M)


def kernel(x, W1, b1, W2, b2):
    attn4 = _attn4(x, W1, b1, W2, b2)
    attn = attn4.mean(axis=1)
    scores = jax.nn.sigmoid(attn)
    k = max(1, int(RATIO * N))
    sorted_scores, sorted_idx = _topk_sort(scores)
    idx = sorted_idx[:k]
    node_feat = jnp.take(x, idx, axis=0)
    scaled_feat = node_feat * (1.0 + sorted_scores[:k][:, None])
    return (scaled_feat, idx, scores)
